# Initial kernel scaffold; baseline (speedup 1.0000x reference)
import jax
import jax.numpy as jnp
from jax.experimental import pallas as pl


def kernel(relative_position_bias_table, relative_position_index):
    raise NotImplementedError("write your pallas kernel here")


# speedup vs baseline: 9.1484x; 9.1484x over previous
import functools

import jax
import jax.numpy as jnp
from jax import lax
from jax.experimental import pallas as pl
from jax.experimental.pallas import tpu as pltpu
from jax.experimental.pallas import tpu_sc as plsc

_L = 16


def _sc_gather(H, V, B, NC, NS):
    NW = NC * NS
    b_per_w = B // NW
    C = 8192
    n_chunks = b_per_w // C

    mesh = plsc.VectorSubcoreMesh(core_axis_name="c", subcore_axis_name="s")

    @functools.partial(
        pl.kernel,
        out_type=jax.ShapeDtypeStruct((H, B), jnp.float32),
        mesh=mesh,
        scratch_types=[
            pltpu.VMEM((H * V,), jnp.float32),
            pltpu.VMEM((b_per_w,), jnp.int32),
            pltpu.VMEM((C,), jnp.float32),
        ],
        compiler_params=pltpu.CompilerParams(needs_layout_passes=False),
    )
    def run(table_hbm, idx_hbm, out_hbm, table_v, idx_v, out_v):
        wid = lax.axis_index("s") * NC + lax.axis_index("c")
        base = wid * b_per_w
        pltpu.sync_copy(table_hbm, table_v)
        pltpu.sync_copy(idx_hbm.at[pl.ds(base, b_per_w)], idx_v)

        def head_body(h, _):
            row = h * V

            def chunk_body(c, _):
                off = c * C

                def vec_body(n, _):
                    iv = idx_v[pl.ds(off + n * _L, _L)]
                    out_v[pl.ds(n * _L, _L)] = plsc.load_gather(
                        table_v, [row + iv])
                    return 0

                lax.fori_loop(0, C // _L, vec_body, 0)
                pltpu.sync_copy(out_v, out_hbm.at[h, pl.ds(base + off, C)])
                return 0

            lax.fori_loop(0, n_chunks, chunk_body, 0)
            return 0

        lax.fori_loop(0, H, head_body, 0)

    return run


def kernel(relative_position_bias_table, relative_position_index):
    V, H = relative_position_bias_table.shape
    S = relative_position_index.shape[0]
    B = S * S
    info = plsc.get_sparse_core_info()
    NC, NS = info.num_cores, info.num_subcores

    tableT = relative_position_bias_table.T.reshape(-1)
    idx_flat = relative_position_index.reshape(-1)

    out = _sc_gather(H, V, B, NC, NS)(tableT, idx_flat)
    return out.reshape(H, S, S)


# speedup vs baseline: 15.8961x; 1.7376x over previous
import functools

import jax
import jax.numpy as jnp
from jax import lax
from jax.experimental import pallas as pl
from jax.experimental.pallas import tpu as pltpu
from jax.experimental.pallas import tpu_sc as plsc

_L = 16


def _sc_gather(H, V, B, NC, NS):
    NW = NC * NS
    b_per_w = B // NW
    C = 2048
    n_chunks = b_per_w // C

    mesh = plsc.VectorSubcoreMesh(core_axis_name="c", subcore_axis_name="s")

    @functools.partial(
        pl.kernel,
        out_type=jax.ShapeDtypeStruct((H, B), jnp.float32),
        mesh=mesh,
        scratch_types=[
            pltpu.VMEM((H * V,), jnp.float32),
            pltpu.VMEM((C,), jnp.int32),
            pltpu.VMEM((H, C), jnp.float32),
        ],
        compiler_params=pltpu.CompilerParams(needs_layout_passes=False),
    )
    def run(table_hbm, idx_hbm, out_hbm, table_v, idx_v, out_v):
        wid = lax.axis_index("s") * NC + lax.axis_index("c")
        base = wid * b_per_w
        pltpu.sync_copy(table_hbm, table_v)

        def chunk_body(c, _):
            off = base + c * C
            pltpu.sync_copy(idx_hbm.at[pl.ds(off, C)], idx_v)

            def vec_body(n, _):
                iv = idx_v[pl.ds(n * _L, _L)]
                for h in range(H):
                    out_v[h, pl.ds(n * _L, _L)] = plsc.load_gather(
                        table_v, [h * V + iv])
                return 0

            lax.fori_loop(0, C // _L, vec_body, 0)
            pltpu.sync_copy(out_v, out_hbm.at[:, pl.ds(off, C)])
            return 0

        lax.fori_loop(0, n_chunks, chunk_body, 0)

    return run


def kernel(relative_position_bias_table, relative_position_index):
    V, H = relative_position_bias_table.shape
    S = relative_position_index.shape[0]
    B = S * S
    info = plsc.get_sparse_core_info()
    NC, NS = info.num_cores, info.num_subcores

    tableT = relative_position_bias_table.T.reshape(-1)
    idx_flat = relative_position_index.reshape(-1)

    out = _sc_gather(H, V, B, NC, NS)(tableT, idx_flat)
    return out.reshape(H, S, S)


# speedup vs baseline: 36.3539x; 2.2870x over previous
import functools

import jax
import jax.numpy as jnp
from jax import lax
from jax.experimental import pallas as pl
from jax.experimental.pallas import tpu as pltpu
from jax.experimental.pallas import tpu_sc as plsc

_L = 16


def _sc_gather(H, V, B, NC, NS):
    NW = NC * NS
    b_per_w = B // NW
    C = 1024
    n_chunks = b_per_w // C
    n_pairs = n_chunks // 2

    mesh = plsc.VectorSubcoreMesh(core_axis_name="c", subcore_axis_name="s")

    @functools.partial(
        pl.kernel,
        out_type=jax.ShapeDtypeStruct((H, B), jnp.float32),
        mesh=mesh,
        scratch_types=[
            pltpu.VMEM((H * V,), jnp.float32),
            pltpu.VMEM((2 * C,), jnp.int32),
            pltpu.VMEM((2 * H, C), jnp.float32),
            pltpu.SemaphoreType.DMA,
            pltpu.SemaphoreType.DMA,
            pltpu.SemaphoreType.DMA,
            pltpu.SemaphoreType.DMA,
        ],
        compiler_params=pltpu.CompilerParams(needs_layout_passes=False),
    )
    def run(table_hbm, idx_hbm, out_hbm, table_v, idx_v, out_v,
            semi0, semi1, semo0, semo1):
        wid = lax.axis_index("s") * NC + lax.axis_index("c")
        base = wid * b_per_w
        semi = (semi0, semi1)
        semo = (semo0, semo1)
        pltpu.sync_copy(table_hbm, table_v)
        pltpu.async_copy(
            idx_hbm.at[pl.ds(base, C)], idx_v.at[pl.ds(0, C)], semi[0])

        def do_chunk(p, par):
            c = 2 * p + par
            off = base + c * C
            nxt = 1 - par

            @pl.when(c + 1 < n_chunks)
            def _():
                pltpu.async_copy(
                    idx_hbm.at[pl.ds(off + C, C)],
                    idx_v.at[pl.ds(nxt * C, C)], semi[nxt])

            @pl.when(p > 0)
            def _():
                pltpu.make_async_copy(
                    out_v.at[pl.ds(par * H, H), :],
                    out_hbm.at[:, pl.ds(off, C)], semo[par]).wait()

            pltpu.make_async_copy(
                idx_hbm.at[pl.ds(off, C)],
                idx_v.at[pl.ds(par * C, C)], semi[par]).wait()

            @plsc.parallel_loop(0, C // _L, unroll=2)
            def _(n):
                iv = idx_v[pl.ds(par * C + n * _L, _L)]
                for h in range(H):
                    out_v[par * H + h, pl.ds(n * _L, _L)] = plsc.load_gather(
                        table_v, [h * V + iv])

            pltpu.async_copy(
                out_v.at[pl.ds(par * H, H), :],
                out_hbm.at[:, pl.ds(off, C)], semo[par])

        def pair_body(p, _):
            do_chunk(p, 0)
            do_chunk(p, 1)
            return 0

        lax.fori_loop(0, n_pairs, pair_body, 0)

        for par in (0, 1):
            pltpu.make_async_copy(
                out_v.at[pl.ds(par * H, H), :],
                out_hbm.at[:, pl.ds(base, C)], semo[par]).wait()

    return run


def kernel(relative_position_bias_table, relative_position_index):
    V, H = relative_position_bias_table.shape
    S = relative_position_index.shape[0]
    B = S * S
    info = plsc.get_sparse_core_info()
    NC, NS = info.num_cores, info.num_subcores

    tableT = relative_position_bias_table.T.reshape(-1)
    idx_flat = relative_position_index.reshape(-1)

    out = _sc_gather(H, V, B, NC, NS)(tableT, idx_flat)
    return out.reshape(H, S, S)


# speedup vs baseline: 53.4229x; 1.4695x over previous
import functools

import jax
import jax.numpy as jnp
from jax import lax
from jax.experimental import pallas as pl
from jax.experimental.pallas import tpu as pltpu
from jax.experimental.pallas import tpu_sc as plsc

_L = 16


def _sc_gather(H, V, S, NC, NS):
    NW = NC * NS
    rows_per_w = S // NW
    RB = 8
    n_ib = rows_per_w // RB
    HP = H // 2
    n_chunks = n_ib * HP

    mesh = plsc.VectorSubcoreMesh(core_axis_name="c", subcore_axis_name="s")

    @functools.partial(
        pl.kernel,
        out_type=jax.ShapeDtypeStruct((H, S, S), jnp.float32),
        mesh=mesh,
        scratch_types=[
            pltpu.VMEM((H * V,), jnp.float32),
            pltpu.VMEM((2 * RB, S), jnp.int32),
            pltpu.VMEM((4 * RB, S), jnp.float32),
            pltpu.SemaphoreType.DMA((2,)),
            pltpu.SemaphoreType.DMA((2,)),
        ],
        compiler_params=pltpu.CompilerParams(
            needs_layout_passes=False, use_tc_tiling_on_sc=True),
    )
    def run(table_hbm, idx_hbm, out_hbm, table_v, idx_v, out_v, semi, semo):
        wid = lax.axis_index("s") * NC + lax.axis_index("c")
        base_i = wid * rows_per_w
        pltpu.sync_copy(table_hbm, table_v)
        pltpu.async_copy(
            idx_hbm.at[pl.ds(base_i, RB), :],
            idx_v.at[pl.ds(0, RB), :], semi.at[0])

        def chunk_body(t, _):
            ib = t // HP
            hp = t - ib * HP
            pi = lax.rem(ib, 2)
            po = lax.rem(t, 2)
            h0 = 2 * hp
            i0 = base_i + ib * RB

            @pl.when(jnp.logical_and(hp == 0, ib + 1 < n_ib))
            def _():
                pltpu.async_copy(
                    idx_hbm.at[pl.ds(i0 + RB, RB), :],
                    idx_v.at[pl.ds((1 - pi) * RB, RB), :], semi.at[1 - pi])

            @pl.when(hp == 0)
            def _():
                pltpu.make_async_copy(
                    idx_hbm.at[pl.ds(i0, RB), :],
                    idx_v.at[pl.ds(pi * RB, RB), :], semi.at[pi]).wait()

            @pl.when(t >= 2)
            def _():
                for _hh in range(2):
                    pltpu.make_async_copy(
                        out_v.at[pl.ds(po * 2 * RB + _hh * RB, RB), :],
                        out_hbm.at[h0 + _hh, pl.ds(i0, RB), :],
                        semo.at[po]).wait()

            ir = pi * RB
            orow = po * 2 * RB
            for r in range(RB):
                @plsc.parallel_loop(0, S // _L, unroll=2)
                def _(n):
                    iv = idx_v[ir + r, pl.ds(n * _L, _L)]
                    out_v[orow + r, pl.ds(n * _L, _L)] = plsc.load_gather(
                        table_v, [h0 * V + iv])
                    out_v[orow + RB + r, pl.ds(n * _L, _L)] = (
                        plsc.load_gather(table_v, [(h0 + 1) * V + iv]))

            for hh in range(2):
                pltpu.async_copy(
                    out_v.at[pl.ds(orow + hh * RB, RB), :],
                    out_hbm.at[h0 + hh, pl.ds(i0, RB), :], semo.at[po])
            return 0

        lax.fori_loop(0, n_chunks, chunk_body, 0)

        for po in range(2):
            for hh in range(2):
                pltpu.make_async_copy(
                    out_v.at[pl.ds(po * 2 * RB + hh * RB, RB), :],
                    out_hbm.at[hh, pl.ds(base_i, RB), :], semo.at[po]).wait()

    return run


def kernel(relative_position_bias_table, relative_position_index):
    V, H = relative_position_bias_table.shape
    S = relative_position_index.shape[0]
    info = plsc.get_sparse_core_info()
    NC, NS = info.num_cores, info.num_subcores

    tableT = relative_position_bias_table.T.reshape(-1)

    return _sc_gather(H, V, S, NC, NS)(tableT, relative_position_index)


# speedup vs baseline: 60.6869x; 1.1360x over previous
import functools

import jax
import jax.numpy as jnp
from jax import lax
from jax.experimental import pallas as pl
from jax.experimental.pallas import tpu as pltpu
from jax.experimental.pallas import tpu_sc as plsc

_L = 16


def _sc_gather(H, V, S, NC, NS):
    NW = NC * NS
    rows_per_w = S // NW
    RB = 8
    n_ib = rows_per_w // RB
    HP = H // 2
    n_chunks = n_ib * HP

    mesh = plsc.VectorSubcoreMesh(core_axis_name="c", subcore_axis_name="s")

    @functools.partial(
        pl.kernel,
        out_type=jax.ShapeDtypeStruct((H, S, S), jnp.float32),
        mesh=mesh,
        scratch_types=[
            pltpu.VMEM((H * V,), jnp.float32),
            pltpu.VMEM((2 * RB, S), jnp.int32),
            pltpu.VMEM((4 * RB, S), jnp.float32),
            pltpu.SemaphoreType.DMA((2,)),
            pltpu.SemaphoreType.DMA((2,)),
        ],
        compiler_params=pltpu.CompilerParams(
            needs_layout_passes=False, use_tc_tiling_on_sc=True),
    )
    def run(table_hbm, idx_hbm, out_hbm, table_v, idx_v, out_v, semi, semo):
        wid = lax.axis_index("s") * NC + lax.axis_index("c")
        base_i = wid * rows_per_w
        pltpu.sync_copy(table_hbm, table_v)
        pltpu.async_copy(
            idx_hbm.at[pl.ds(base_i, RB), :],
            idx_v.at[pl.ds(0, RB), :], semi.at[0])

        def chunk_body(t, _):
            ib = t // HP
            hp = t - ib * HP
            pi = lax.rem(ib, 2)
            po = lax.rem(t, 2)
            h0 = 2 * hp
            i0 = base_i + ib * RB

            @pl.when(jnp.logical_and(hp == 0, ib + 1 < n_ib))
            def _():
                pltpu.async_copy(
                    idx_hbm.at[pl.ds(i0 + RB, RB), :],
                    idx_v.at[pl.ds((1 - pi) * RB, RB), :], semi.at[1 - pi])

            @pl.when(hp == 0)
            def _():
                pltpu.make_async_copy(
                    idx_hbm.at[pl.ds(i0, RB), :],
                    idx_v.at[pl.ds(pi * RB, RB), :], semi.at[pi]).wait()

            @pl.when(t >= 2)
            def _():
                for _hh in range(2):
                    pltpu.make_async_copy(
                        out_v.at[pl.ds(po * 2 * RB + _hh * RB, RB), :],
                        out_hbm.at[h0 + _hh, pl.ds(i0, RB), :],
                        semo.at[po]).wait()

            ir = pi * RB
            orow = po * 2 * RB
            for r in range(RB):
                @plsc.parallel_loop(0, S // _L, unroll=4)
                def _(n):
                    iv = idx_v[ir + r, pl.ds(n * _L, _L)]
                    out_v[orow + r, pl.ds(n * _L, _L)] = plsc.load_gather(
                        table_v, [h0 * V + iv])
                    out_v[orow + RB + r, pl.ds(n * _L, _L)] = (
                        plsc.load_gather(table_v, [(h0 + 1) * V + iv]))

            for hh in range(2):
                pltpu.async_copy(
                    out_v.at[pl.ds(orow + hh * RB, RB), :],
                    out_hbm.at[h0 + hh, pl.ds(i0, RB), :], semo.at[po])
            return 0

        lax.fori_loop(0, n_chunks, chunk_body, 0)

        for po in range(2):
            for hh in range(2):
                pltpu.make_async_copy(
                    out_v.at[pl.ds(po * 2 * RB + hh * RB, RB), :],
                    out_hbm.at[hh, pl.ds(base_i, RB), :], semo.at[po]).wait()

    return run


def kernel(relative_position_bias_table, relative_position_index):
    V, H = relative_position_bias_table.shape
    S = relative_position_index.shape[0]
    info = plsc.get_sparse_core_info()
    NC, NS = info.num_cores, info.num_subcores

    tableT = relative_position_bias_table.T.reshape(-1)

    return _sc_gather(H, V, S, NC, NS)(tableT, relative_position_index)


# speedup vs baseline: 67.8813x; 1.1185x over previous
import functools

import jax
import jax.numpy as jnp
from jax import lax
from jax.experimental import pallas as pl
from jax.experimental.pallas import tpu as pltpu
from jax.experimental.pallas import tpu_sc as plsc

_L = 16


def _sc_gather(H, V, S, NC, NS):
    NW = NC * NS
    rows_per_w = S // NW
    RB = 8
    n_ib = rows_per_w // RB
    HP = H // 2
    n_chunks = n_ib * HP

    mesh = plsc.VectorSubcoreMesh(core_axis_name="c", subcore_axis_name="s")

    @functools.partial(
        pl.kernel,
        out_type=jax.ShapeDtypeStruct((H, S, S), jnp.float32),
        mesh=mesh,
        scratch_types=[
            pltpu.VMEM((H * V,), jnp.float32),
            pltpu.VMEM((2 * RB, S), jnp.int32),
            pltpu.VMEM((4 * RB, S), jnp.float32),
            pltpu.SemaphoreType.DMA((2,)),
            pltpu.SemaphoreType.DMA((2,)),
        ],
        compiler_params=pltpu.CompilerParams(
            needs_layout_passes=False, use_tc_tiling_on_sc=True),
    )
    def run(table_hbm, idx_hbm, out_hbm, table_v, idx_v, out_v, semi, semo):
        wid = lax.axis_index("s") * NC + lax.axis_index("c")
        base_i = wid * rows_per_w
        pltpu.sync_copy(table_hbm, table_v)
        pltpu.async_copy(
            idx_hbm.at[pl.ds(base_i, RB), :],
            idx_v.at[pl.ds(0, RB), :], semi.at[0])

        def chunk_body(t, _):
            ib = t // HP
            hp = t - ib * HP
            pi = lax.rem(ib, 2)
            po = lax.rem(t, 2)
            h0 = 2 * hp
            i0 = base_i + ib * RB

            @pl.when(jnp.logical_and(hp == 0, ib + 1 < n_ib))
            def _():
                pltpu.async_copy(
                    idx_hbm.at[pl.ds(i0 + RB, RB), :],
                    idx_v.at[pl.ds((1 - pi) * RB, RB), :], semi.at[1 - pi])

            @pl.when(hp == 0)
            def _():
                pltpu.make_async_copy(
                    idx_hbm.at[pl.ds(i0, RB), :],
                    idx_v.at[pl.ds(pi * RB, RB), :], semi.at[pi]).wait()

            @pl.when(t >= 2)
            def _():
                for _hh in range(2):
                    pltpu.make_async_copy(
                        out_v.at[pl.ds(po * 2 * RB + _hh * RB, RB), :],
                        out_hbm.at[h0 + _hh, pl.ds(i0, RB), :],
                        semo.at[po]).wait()

            ir = pi * RB
            orow = po * 2 * RB
            for r in range(RB):
                @plsc.parallel_loop(0, S // _L, unroll=8)
                def _(n):
                    iv = idx_v[ir + r, pl.ds(n * _L, _L)]
                    out_v[orow + r, pl.ds(n * _L, _L)] = plsc.load_gather(
                        table_v, [h0 * V + iv])
                    out_v[orow + RB + r, pl.ds(n * _L, _L)] = (
                        plsc.load_gather(table_v, [(h0 + 1) * V + iv]))

            for hh in range(2):
                pltpu.async_copy(
                    out_v.at[pl.ds(orow + hh * RB, RB), :],
                    out_hbm.at[h0 + hh, pl.ds(i0, RB), :], semo.at[po])
            return 0

        lax.fori_loop(0, n_chunks, chunk_body, 0)

        for po in range(2):
            for hh in range(2):
                pltpu.make_async_copy(
                    out_v.at[pl.ds(po * 2 * RB + hh * RB, RB), :],
                    out_hbm.at[hh, pl.ds(base_i, RB), :], semo.at[po]).wait()

    return run


def kernel(relative_position_bias_table, relative_position_index):
    V, H = relative_position_bias_table.shape
    S = relative_position_index.shape[0]
    info = plsc.get_sparse_core_info()
    NC, NS = info.num_cores, info.num_subcores

    tableT = relative_position_bias_table.T.reshape(-1)

    return _sc_gather(H, V, S, NC, NS)(tableT, relative_position_index)
